# baseline (speedup 1.0000x reference)
import functools
import math

import jax
import jax.numpy as jnp
from jax import lax
from jax.experimental import pallas as pl
from jax.experimental.pallas import tpu as pltpu
from jax.experimental.pallas import tpu_sc as plsc

B, T, D, E = 2, 2048, 1024, 8
THRESHOLD = 0.8
CAP_FACTOR_EVAL = 2.0
MIN_EXPERT_CAPACITY = 4
C = max(MIN_EXPERT_CAPACITY, min(T, math.ceil(T * CAP_FACTOR_EVAL / E)))

TT = 512
TOK = B * T
NW = 32
TPW = TOK // NW
CH = 8
NCH = TPW // CH
LANES = 16


def _meta_kernel(x_ref, w_ref, posx_ref, wts_ref, aux_ref,
                 carry_ref, accm_ref, accp_ref, auxacc_ref):
    b = pl.program_id(0)
    t = pl.program_id(1)
    n_t = pl.num_programs(1)

    @pl.when(t == 0)
    def _():
        carry_ref[...] = jnp.zeros_like(carry_ref)
        accm_ref[...] = jnp.zeros_like(accm_ref)
        accp_ref[...] = jnp.zeros_like(accp_ref)

    @pl.when((b == 0) & (t == 0))
    def _():
        auxacc_ref[...] = jnp.zeros_like(auxacc_ref)

    xb = x_ref[0]
    gates = jnp.dot(xb, w_ref[...], preferred_element_type=jnp.float32)
    m = jnp.max(gates, axis=-1, keepdims=True)
    ex = jnp.exp(gates - m)
    probs = ex / jnp.sum(ex, axis=-1, keepdims=True)

    s_incl = probs
    for r in range(1, E):
        pr = jnp.roll(probs, r, axis=1)
        s_incl = s_incl + jnp.where(pr >= probs, pr, 0.0)
    pmax = jnp.max(probs, axis=-1, keepdims=True)
    sel = (s_incl < THRESHOLD) | (probs >= pmax)
    maskf = sel.astype(jnp.float32)
    selp = probs * maskf
    renorm = jnp.maximum(jnp.sum(selp, axis=-1, keepdims=True), 1e-9)
    weights = selp / renorm

    row = jax.lax.broadcasted_iota(jnp.int32, (TT, TT), 0)
    col = jax.lax.broadcasted_iota(jnp.int32, (TT, TT), 1)
    ltri = (col < row).astype(jnp.float32)
    pos = jnp.dot(ltri, maskf, preferred_element_type=jnp.float32)
    pos = pos + carry_ref[0][None, :]
    carry_ref[0] = carry_ref[0] + jnp.sum(maskf, axis=0)

    accm_ref[0] = accm_ref[0] + jnp.sum(maskf, axis=0)
    accp_ref[0] = accp_ref[0] + jnp.sum(probs, axis=0)

    kc = (pos < float(C)) & sel
    posx_ref[0] = jnp.where(kc, pos, float(C)).astype(jnp.int32)
    wts_ref[0] = weights

    @pl.when(t == n_t - 1)
    def _():
        auxacc_ref[...] = auxacc_ref[...] + jnp.sum(
            accm_ref[...] * accp_ref[...], axis=(0, 1), keepdims=True)

    @pl.when((b == B - 1) & (t == n_t - 1))
    def _():
        aux_ref[...] = auxacc_ref[...] * (float(E) / (float(B) * float(T) * float(T)))


def _disp_kernel(posx_ref, disp_ref):
    posx = posx_ref[0]
    ciota = jax.lax.broadcasted_iota(jnp.int32, (TT, E, C), 2)
    disp_ref[0] = (ciota == posx[:, :, None]).astype(jnp.float32)


_SC_MESH = plsc.VectorSubcoreMesh(core_axis_name="c", subcore_axis_name="s")


@functools.partial(
    pl.kernel,
    out_type=jax.ShapeDtypeStruct((TOK * E * C,), jnp.float32),
    mesh=_SC_MESH,
    scratch_types=[
        pltpu.VMEM((TPW * E,), jnp.int32),
        pltpu.VMEM((TPW * E,), jnp.float32),
        pltpu.VMEM((CH * E * C,), jnp.float32),
    ],
)
def _comb_sc(posx_hbm, wts_hbm, out_hbm, pos_v, wts_v, buf):
    wid = lax.axis_index("s") * 2 + lax.axis_index("c")
    base_e = wid * (TPW * E)
    base_t = wid * TPW

    pltpu.sync_copy(posx_hbm.at[pl.ds(base_e, TPW * E)], pos_v)
    pltpu.sync_copy(wts_hbm.at[pl.ds(base_e, TPW * E)], wts_v)

    zero16 = jnp.zeros((LANES,), jnp.float32)

    def _zbody(i, _):
        buf[pl.ds(i * LANES, LANES)] = zero16
        return 0
    lax.fori_loop(0, (CH * E * C) // LANES, _zbody, 0)

    iota16 = lax.broadcasted_iota(jnp.int32, (LANES,), 0)

    def _entry(ent, p, w, value16_fn):
        tok, e = ent // E, ent % E
        valid = p < C
        pc = jnp.where(valid, p, 0)
        base16 = tok * (E * C) + e * C + ((pc >> 4) << 4)
        lane = jnp.where(valid, pc & 15, LANES)
        v = buf[pl.ds(base16, LANES)]
        m = iota16 == lane
        buf[pl.ds(base16, LANES)] = jnp.where(m, value16_fn(w), v)

    def _pass(k, value16_fn):
        for g in range(CH * E // LANES):
            p16 = pos_v[pl.ds(k * (CH * E) + g * LANES, LANES)]
            w16 = wts_v[pl.ds(k * (CH * E) + g * LANES, LANES)]
            for j in range(LANES):
                _entry(g * LANES + j, p16[j], w16[j], value16_fn)

    def _chunk(k, _):
        _pass(k, lambda w: jnp.full((LANES,), w, jnp.float32))
        pltpu.sync_copy(
            buf, out_hbm.at[pl.ds((base_t + k * CH) * (E * C), CH * E * C)])
        _pass(k, lambda w: jnp.zeros((LANES,), jnp.float32))
        return 0

    lax.fori_loop(0, NCH, _chunk, 0)


@jax.jit
def kernel(x, w_gating):
    n_t = T // TT
    posx, wts, aux = pl.pallas_call(
        _meta_kernel,
        grid=(B, n_t),
        in_specs=[
            pl.BlockSpec((1, TT, D), lambda b, t: (b, t, 0)),
            pl.BlockSpec((D, E), lambda b, t: (0, 0)),
        ],
        out_specs=[
            pl.BlockSpec((1, TT, E), lambda b, t: (b, t, 0)),
            pl.BlockSpec((1, TT, E), lambda b, t: (b, t, 0)),
            pl.BlockSpec((1, 1), lambda b, t: (0, 0)),
        ],
        out_shape=[
            jax.ShapeDtypeStruct((B, T, E), jnp.int32),
            jax.ShapeDtypeStruct((B, T, E), jnp.float32),
            jax.ShapeDtypeStruct((1, 1), jnp.float32),
        ],
        scratch_shapes=[
            pltpu.VMEM((1, E), jnp.float32),
            pltpu.VMEM((1, E), jnp.float32),
            pltpu.VMEM((1, E), jnp.float32),
            pltpu.VMEM((1, 1), jnp.float32),
        ],
    )(x, w_gating)

    disp = pl.pallas_call(
        _disp_kernel,
        grid=(B, n_t),
        in_specs=[pl.BlockSpec((1, TT, E), lambda b, t: (b, t, 0))],
        out_specs=[pl.BlockSpec((1, TT, E, C), lambda b, t: (b, t, 0, 0))],
        out_shape=[jax.ShapeDtypeStruct((B, T, E, C), jnp.float32)],
    )(posx)[0]

    comb = _comb_sc(posx.reshape(TOK * E), wts.reshape(TOK * E))
    return disp, comb.reshape(B, T, E, C), aux[0, 0]


# speedup vs baseline: 3.2755x; 3.2755x over previous
import functools
import math

import jax
import jax.numpy as jnp
from jax.experimental import pallas as pl
from jax.experimental.pallas import tpu as pltpu

B, T, D, E = 2, 2048, 1024, 8
THRESHOLD = 0.8
CAP_FACTOR_EVAL = 2.0
MIN_EXPERT_CAPACITY = 4
C = max(MIN_EXPERT_CAPACITY, min(T, math.ceil(T * CAP_FACTOR_EVAL / E)))

TT = 512


def _gating_kernel(x_ref, w_ref, disp_ref, comb_ref, aux_ref,
                   carry_ref, accm_ref, accp_ref, auxacc_ref):
    b = pl.program_id(0)
    t = pl.program_id(1)
    n_t = pl.num_programs(1)

    @pl.when(t == 0)
    def _():
        carry_ref[...] = jnp.zeros_like(carry_ref)
        accm_ref[...] = jnp.zeros_like(accm_ref)
        accp_ref[...] = jnp.zeros_like(accp_ref)

    @pl.when((b == 0) & (t == 0))
    def _():
        auxacc_ref[...] = jnp.zeros_like(auxacc_ref)

    xb = x_ref[0]
    gates = jnp.dot(xb, w_ref[...], preferred_element_type=jnp.float32)
    m = jnp.max(gates, axis=-1, keepdims=True)
    ex = jnp.exp(gates - m)
    probs = ex / jnp.sum(ex, axis=-1, keepdims=True)

    s_incl = probs
    for r in range(1, E):
        pr = jnp.roll(probs, r, axis=1)
        s_incl = s_incl + jnp.where(pr >= probs, pr, 0.0)
    pmax = jnp.max(probs, axis=-1, keepdims=True)
    sel = (s_incl < THRESHOLD) | (probs >= pmax)
    maskf = sel.astype(jnp.float32)
    selp = probs * maskf
    renorm = jnp.maximum(jnp.sum(selp, axis=-1, keepdims=True), 1e-9)
    weights = selp / renorm

    row = jax.lax.broadcasted_iota(jnp.int32, (TT, TT), 0)
    col = jax.lax.broadcasted_iota(jnp.int32, (TT, TT), 1)
    ltri = (col < row).astype(jnp.float32)
    pos = jnp.dot(ltri, maskf, preferred_element_type=jnp.float32)
    pos = pos + carry_ref[0][None, :]
    carry_ref[0] = carry_ref[0] + jnp.sum(maskf, axis=0)

    accm_ref[0] = accm_ref[0] + jnp.sum(maskf, axis=0)
    accp_ref[0] = accp_ref[0] + jnp.sum(probs, axis=0)

    kc = (pos < float(C)) & sel
    posx = jnp.where(kc, pos, float(C)).astype(jnp.int32)
    ciota = jax.lax.broadcasted_iota(jnp.int32, (TT, E, C), 2)
    onehot = (ciota == posx[:, :, None]).astype(jnp.float32)
    disp_ref[0] = onehot
    comb_ref[0] = onehot * weights[:, :, None]

    @pl.when(t == n_t - 1)
    def _():
        auxacc_ref[...] = auxacc_ref[...] + jnp.sum(
            accm_ref[...] * accp_ref[...], axis=(0, 1), keepdims=True)

    @pl.when((b == B - 1) & (t == n_t - 1))
    def _():
        aux_ref[...] = auxacc_ref[...] * (float(E) / (float(B) * float(T) * float(T)))


@jax.jit
def kernel(x, w_gating):
    n_t = T // TT
    disp, comb, aux = pl.pallas_call(
        _gating_kernel,
        grid=(B, n_t),
        in_specs=[
            pl.BlockSpec((1, TT, D), lambda b, t: (b, t, 0)),
            pl.BlockSpec((D, E), lambda b, t: (0, 0)),
        ],
        out_specs=[
            pl.BlockSpec((1, TT, E, C), lambda b, t: (b, t, 0, 0)),
            pl.BlockSpec((1, TT, E, C), lambda b, t: (b, t, 0, 0)),
            pl.BlockSpec((1, 1), lambda b, t: (0, 0)),
        ],
        out_shape=[
            jax.ShapeDtypeStruct((B, T, E, C), jnp.float32),
            jax.ShapeDtypeStruct((B, T, E, C), jnp.float32),
            jax.ShapeDtypeStruct((1, 1), jnp.float32),
        ],
        scratch_shapes=[
            pltpu.VMEM((1, E), jnp.float32),
            pltpu.VMEM((1, E), jnp.float32),
            pltpu.VMEM((1, E), jnp.float32),
            pltpu.VMEM((1, 1), jnp.float32),
        ],
    )(x, w_gating)
    return disp, comb, aux[0, 0]
